# initial kernel scaffold (unmeasured)
import jax
import jax.numpy as jnp
from jax import lax
from jax.experimental import pallas as pl
from jax.experimental.pallas import tpu as pltpu

N_DEV = 16
S = 4096
M = 256
D = 1024
HL = 8
DH = 128
QB = 512
SCALE = 0.08838834764831843


def kernel(x, Wq, Wo, Wk, Wv):
    def body(x_ref, wq_ref, wo_ref, wk_ref, wv_ref, out_ref,
             x_full, o_full, send_buf, rs_buf,
             ag_send_sems, ag_recv_sems, rs_send_sems, rs_recv_sems,
             credit_sem):
        my = lax.axis_index("i")
        left = jnp.mod(my - 1, N_DEV)
        right = jnp.mod(my + 1, N_DEV)

        barrier = pltpu.get_barrier_semaphore()
        for nbr in (left, right):
            pl.semaphore_signal(barrier, inc=1, device_id=(nbr,),
                                device_id_type=pl.DeviceIdType.MESH)
        pl.semaphore_wait(barrier, 2)

        x_full[pl.ds(my * M, M), :] = x_ref[0]

        for h in range(N_DEV - 1):
            c_send = jnp.mod(my - h, N_DEV)
            c_recv = jnp.mod(my - h - 1, N_DEV)
            send = pltpu.make_async_remote_copy(
                src_ref=x_full.at[pl.ds(c_send * M, M), :],
                dst_ref=x_full.at[pl.ds(c_send * M, M), :],
                send_sem=ag_send_sems.at[h],
                recv_sem=ag_recv_sems.at[h],
                device_id=(right,),
                device_id_type=pl.DeviceIdType.MESH,
            )
            send.start()
            send.wait_send()
            recv = pltpu.make_async_remote_copy(
                src_ref=x_full.at[pl.ds(c_recv * M, M), :],
                dst_ref=x_full.at[pl.ds(c_recv * M, M), :],
                send_sem=ag_send_sems.at[h],
                recv_sem=ag_recv_sems.at[h],
                device_id=(left,),
                device_id_type=pl.DeviceIdType.MESH,
            )
            recv.wait_recv()

        x2d = x_full[...]
        for hh in range(HL):
            wk_h = wk_ref[:, hh * DH:(hh + 1) * DH]
            wv_h = wv_ref[:, hh * DH:(hh + 1) * DH]
            wq_h = wq_ref[:, hh * DH:(hh + 1) * DH]
            k_h = jnp.dot(x2d, wk_h, preferred_element_type=jnp.float32)
            v_h = jnp.dot(x2d, wv_h, preferred_element_type=jnp.float32)
            for qb in range(S // QB):
                xq = x2d[qb * QB:(qb + 1) * QB, :]
                q = jnp.dot(xq, wq_h, preferred_element_type=jnp.float32)
                s = lax.dot_general(
                    q, k_h, (((1,), (1,)), ((), ())),
                    preferred_element_type=jnp.float32) * SCALE
                m_ = jnp.max(s, axis=-1, keepdims=True)
                p_ = jnp.exp(s - m_)
                l_ = jnp.sum(p_, axis=-1, keepdims=True)
                o = jnp.dot(p_, v_h, preferred_element_type=jnp.float32) / l_
                o_full[qb * QB:(qb + 1) * QB, hh * DH:(hh + 1) * DH] = o

        wo = wo_ref[...]
        for t in range(N_DEV - 1):
            c = jnp.mod(my - 1 - t, N_DEV)
            oc = o_full[pl.ds(c * M, M), :]
            pc = jnp.dot(oc, wo, preferred_element_type=jnp.float32)
            if t == 0:
                payload = pc
            else:
                payload = rs_buf[(t - 1) % 2] + pc
            send_buf[t % 2] = payload
            if t >= 2:
                pl.semaphore_wait(credit_sem, 1)
            send = pltpu.make_async_remote_copy(
                src_ref=send_buf.at[t % 2],
                dst_ref=rs_buf.at[t % 2],
                send_sem=rs_send_sems.at[t],
                recv_sem=rs_recv_sems.at[t],
                device_id=(right,),
                device_id_type=pl.DeviceIdType.MESH,
            )
            send.start()
            send.wait_send()
            recv = pltpu.make_async_remote_copy(
                src_ref=send_buf.at[t % 2],
                dst_ref=rs_buf.at[t % 2],
                send_sem=rs_send_sems.at[t],
                recv_sem=rs_recv_sems.at[t],
                device_id=(left,),
                device_id_type=pl.DeviceIdType.MESH,
            )
            recv.wait_recv()
            pl.semaphore_signal(credit_sem, inc=1, device_id=(left,),
                                device_id_type=pl.DeviceIdType.MESH)

        oc = o_full[pl.ds(my * M, M), :]
        pc = jnp.dot(oc, wo, preferred_element_type=jnp.float32)
        out_ref[0] = rs_buf[(N_DEV - 2) % 2] + pc

    return pl.pallas_call(
        body,
        out_shape=jax.ShapeDtypeStruct((1, M, D), jnp.float32),
        in_specs=[pl.BlockSpec(memory_space=pltpu.VMEM)] * 5,
        out_specs=pl.BlockSpec(memory_space=pltpu.VMEM),
        scratch_shapes=[
            pltpu.VMEM((S, D), jnp.float32),
            pltpu.VMEM((S, D), jnp.float32),
            pltpu.VMEM((2, M, D), jnp.float32),
            pltpu.VMEM((2, M, D), jnp.float32),
            pltpu.SemaphoreType.DMA((N_DEV - 1,)),
            pltpu.SemaphoreType.DMA((N_DEV - 1,)),
            pltpu.SemaphoreType.DMA((N_DEV - 1,)),
            pltpu.SemaphoreType.DMA((N_DEV - 1,)),
            pltpu.SemaphoreType.REGULAR,
        ],
        compiler_params=pltpu.CompilerParams(collective_id=0),
    )(x, Wq, Wo, Wk, Wv)


# baseline (device time: 942493 ns/iter reference)
import jax
import jax.numpy as jnp
from jax import lax
from jax.experimental import pallas as pl
from jax.experimental.pallas import tpu as pltpu

N_DEV = 16
S = 4096
M = 256
D = 1024
HL = 8
DH = 128
SCALE = 0.08838834764831843


def kernel(x, Wq, Wo, Wk, Wv):
    def body(x_ref, wq_hbm, wo_hbm, wk_hbm, wv_hbm, out_ref,
             x_full, p_acc, k_scr, v_scr, wq_h, wk_h, wv_h, wo_scr,
             comm, rs_send, rs_recv,
             ag_send_sems, ag_recv_sems, rs_send_sems, rs_recv_sems,
             copy_sem):
        my = lax.axis_index("i")
        left = jnp.mod(my - 1, N_DEV)
        right = jnp.mod(my + 1, N_DEV)

        wo_copy = pltpu.make_async_copy(wo_hbm, wo_scr, copy_sem)
        wo_copy.start()
        wo_copy.wait()

        barrier = pltpu.get_barrier_semaphore()
        for nbr in (left, right):
            pl.semaphore_signal(barrier, inc=1, device_id=(nbr,),
                                device_id_type=pl.DeviceIdType.MESH)
        pl.semaphore_wait(barrier, 2)

        x_full[pl.ds(my * M, M), :] = x_ref[0]
        comm[0] = x_ref[0]

        for h in range(N_DEV - 1):
            s_slot = h % 2
            r_slot = (h + 1) % 2
            rdma = pltpu.make_async_remote_copy(
                src_ref=comm.at[s_slot],
                dst_ref=comm.at[r_slot],
                send_sem=ag_send_sems.at[s_slot],
                recv_sem=ag_recv_sems.at[r_slot],
                device_id=(right,),
                device_id_type=pl.DeviceIdType.MESH,
            )
            rdma.start()
            rdma.wait()
            origin = jnp.mod(my - h - 1, N_DEV)
            x_full[pl.ds(origin * M, M), :] = comm[r_slot]

        for hh in range(HL):
            for whbm, wscr in ((wq_hbm, wq_h), (wk_hbm, wk_h),
                               (wv_hbm, wv_h)):
                c = pltpu.make_async_copy(
                    whbm.at[:, hh * DH:(hh + 1) * DH], wscr, copy_sem)
                c.start()
                c.wait()

            def kv_block(rb, _):
                xb = x_full[pl.ds(rb * M, M), :]
                k_scr[pl.ds(rb * M, M), :] = jnp.dot(
                    xb, wk_h[...], preferred_element_type=jnp.float32)
                v_scr[pl.ds(rb * M, M), :] = jnp.dot(
                    xb, wv_h[...], preferred_element_type=jnp.float32)
                return 0

            lax.fori_loop(0, N_DEV, kv_block, 0)

            def qb_block(qb, _, first=(hh == 0), h0=hh * DH):
                xq = x_full[pl.ds(qb * M, M), :]
                q = jnp.dot(xq, wq_h[...],
                            preferred_element_type=jnp.float32)
                s = lax.dot_general(
                    q, k_scr[...], (((1,), (1,)), ((), ())),
                    preferred_element_type=jnp.float32) * SCALE
                m_ = jnp.max(s, axis=-1, keepdims=True)
                p_ = jnp.exp(s - m_)
                l_ = jnp.sum(p_, axis=-1, keepdims=True)
                o = jnp.dot(p_, v_scr[...],
                            preferred_element_type=jnp.float32) / l_
                upd = jnp.dot(o, wo_scr[h0:h0 + DH, :],
                              preferred_element_type=jnp.float32)
                if first:
                    p_acc[pl.ds(qb * M, M), :] = upd
                else:
                    p_acc[pl.ds(qb * M, M), :] = (
                        p_acc[pl.ds(qb * M, M), :] + upd)
                return 0

            lax.fori_loop(0, N_DEV, qb_block, 0)

        for t in range(N_DEV - 1):
            c = jnp.mod(my - 1 - t, N_DEV)
            pc = p_acc[pl.ds(c * M, M), :]
            if t == 0:
                payload = pc
            else:
                payload = rs_recv[(t - 1) % 2] + pc
            rs_send[t % 2] = payload
            rdma = pltpu.make_async_remote_copy(
                src_ref=rs_send.at[t % 2],
                dst_ref=rs_recv.at[t % 2],
                send_sem=rs_send_sems.at[t % 2],
                recv_sem=rs_recv_sems.at[t % 2],
                device_id=(right,),
                device_id_type=pl.DeviceIdType.MESH,
            )
            rdma.start()
            rdma.wait()

        out_ref[0] = (rs_recv[(N_DEV - 2) % 2]
                      + p_acc[pl.ds(my * M, M), :])

    return pl.pallas_call(
        body,
        out_shape=jax.ShapeDtypeStruct((1, M, D), jnp.float32),
        in_specs=[
            pl.BlockSpec(memory_space=pltpu.VMEM),
            pl.BlockSpec(memory_space=pltpu.MemorySpace.HBM),
            pl.BlockSpec(memory_space=pltpu.MemorySpace.HBM),
            pl.BlockSpec(memory_space=pltpu.MemorySpace.HBM),
            pl.BlockSpec(memory_space=pltpu.MemorySpace.HBM),
        ],
        out_specs=pl.BlockSpec(memory_space=pltpu.VMEM),
        scratch_shapes=[
            pltpu.VMEM((S, D), jnp.float32),
            pltpu.VMEM((S, D), jnp.float32),
            pltpu.VMEM((S, DH), jnp.float32),
            pltpu.VMEM((S, DH), jnp.float32),
            pltpu.VMEM((D, DH), jnp.float32),
            pltpu.VMEM((D, DH), jnp.float32),
            pltpu.VMEM((D, DH), jnp.float32),
            pltpu.VMEM((D, D), jnp.float32),
            pltpu.VMEM((2, M, D), jnp.float32),
            pltpu.VMEM((2, M, D), jnp.float32),
            pltpu.VMEM((2, M, D), jnp.float32),
            pltpu.SemaphoreType.DMA((2,)),
            pltpu.SemaphoreType.DMA((2,)),
            pltpu.SemaphoreType.DMA((2,)),
            pltpu.SemaphoreType.DMA((2,)),
            pltpu.SemaphoreType.DMA,
        ],
        compiler_params=pltpu.CompilerParams(
            collective_id=0,
            vmem_limit_bytes=63 * 1024 * 1024,
        ),
    )(x, Wq, Wo, Wk, Wv)


# device time: 826883 ns/iter; 1.1398x vs baseline; 1.1398x over previous
import jax
import jax.numpy as jnp
from jax import lax
from jax.experimental import pallas as pl
from jax.experimental.pallas import tpu as pltpu

N_DEV = 16
S = 4096
M = 256
D = 1024
HL = 8
DH = 128
SCALE = 0.08838834764831843


def _bdot(a, b, dims=None):
    a16 = a.astype(jnp.bfloat16)
    b16 = b.astype(jnp.bfloat16)
    if dims is None:
        dims = (((a.ndim - 1,), (0,)), ((), ()))
    return lax.dot_general(a16, b16, dims,
                           preferred_element_type=jnp.float32)


def kernel(x, Wq, Wo, Wk, Wv):
    def body(x_ref, wq_hbm, wo_hbm, wk_hbm, wv_hbm, out_ref,
             x_full, p_acc, k_scr, v_scr, wq_h, wk_h, wv_h, wo_scr,
             comm, rs_send, rs_recv,
             ag_send_sems, ag_recv_sems, rs_send_sems, rs_recv_sems,
             copy_sem):
        my = lax.axis_index("i")
        left = jnp.mod(my - 1, N_DEV)
        right = jnp.mod(my + 1, N_DEV)

        wo_copy = pltpu.make_async_copy(wo_hbm, wo_scr, copy_sem)
        wo_copy.start()
        wo_copy.wait()

        barrier = pltpu.get_barrier_semaphore()
        for nbr in (left, right):
            pl.semaphore_signal(barrier, inc=1, device_id=(nbr,),
                                device_id_type=pl.DeviceIdType.MESH)
        pl.semaphore_wait(barrier, 2)

        x_full[pl.ds(my * M, M), :] = x_ref[0]
        comm[0] = x_ref[0]

        for h in range(N_DEV - 1):
            s_slot = h % 2
            r_slot = (h + 1) % 2
            rdma = pltpu.make_async_remote_copy(
                src_ref=comm.at[s_slot],
                dst_ref=comm.at[r_slot],
                send_sem=ag_send_sems.at[s_slot],
                recv_sem=ag_recv_sems.at[r_slot],
                device_id=(right,),
                device_id_type=pl.DeviceIdType.MESH,
            )
            rdma.start()
            rdma.wait()
            origin = jnp.mod(my - h - 1, N_DEV)
            x_full[pl.ds(origin * M, M), :] = comm[r_slot]

        for hh in range(HL):
            for whbm, wscr in ((wq_hbm, wq_h), (wk_hbm, wk_h),
                               (wv_hbm, wv_h)):
                c = pltpu.make_async_copy(
                    whbm.at[:, hh * DH:(hh + 1) * DH], wscr, copy_sem)
                c.start()
                c.wait()

            def kv_block(rb, _):
                xb = x_full[pl.ds(rb * M, M), :]
                k_scr[pl.ds(rb * M, M), :] = _bdot(xb, wk_h[...])
                v_scr[pl.ds(rb * M, M), :] = _bdot(xb, wv_h[...])
                return 0

            lax.fori_loop(0, N_DEV, kv_block, 0)

            def qb_block(qb, _, first=(hh == 0), h0=hh * DH):
                xq = x_full[pl.ds(qb * M, M), :]
                q = _bdot(xq, wq_h[...])
                s = _bdot(q, k_scr[...],
                          (((1,), (1,)), ((), ()))) * SCALE
                m_ = jnp.max(s, axis=-1, keepdims=True)
                p_ = jnp.exp(s - m_)
                l_ = jnp.sum(p_, axis=-1, keepdims=True)
                o = _bdot(p_, v_scr[...]) / l_
                upd = _bdot(o, wo_scr[h0:h0 + DH, :])
                if first:
                    p_acc[pl.ds(qb * M, M), :] = upd
                else:
                    p_acc[pl.ds(qb * M, M), :] = (
                        p_acc[pl.ds(qb * M, M), :] + upd)
                return 0

            lax.fori_loop(0, N_DEV, qb_block, 0)

        for t in range(N_DEV - 1):
            c = jnp.mod(my - 1 - t, N_DEV)
            pc = p_acc[pl.ds(c * M, M), :]
            if t == 0:
                payload = pc
            else:
                payload = rs_recv[(t - 1) % 2] + pc
            rs_send[t % 2] = payload
            rdma = pltpu.make_async_remote_copy(
                src_ref=rs_send.at[t % 2],
                dst_ref=rs_recv.at[t % 2],
                send_sem=rs_send_sems.at[t % 2],
                recv_sem=rs_recv_sems.at[t % 2],
                device_id=(right,),
                device_id_type=pl.DeviceIdType.MESH,
            )
            rdma.start()
            rdma.wait()

        out_ref[0] = (rs_recv[(N_DEV - 2) % 2]
                      + p_acc[pl.ds(my * M, M), :])

    return pl.pallas_call(
        body,
        out_shape=jax.ShapeDtypeStruct((1, M, D), jnp.float32),
        in_specs=[
            pl.BlockSpec(memory_space=pltpu.VMEM),
            pl.BlockSpec(memory_space=pltpu.MemorySpace.HBM),
            pl.BlockSpec(memory_space=pltpu.MemorySpace.HBM),
            pl.BlockSpec(memory_space=pltpu.MemorySpace.HBM),
            pl.BlockSpec(memory_space=pltpu.MemorySpace.HBM),
        ],
        out_specs=pl.BlockSpec(memory_space=pltpu.VMEM),
        scratch_shapes=[
            pltpu.VMEM((S, D), jnp.float32),
            pltpu.VMEM((S, D), jnp.float32),
            pltpu.VMEM((S, DH), jnp.float32),
            pltpu.VMEM((S, DH), jnp.float32),
            pltpu.VMEM((D, DH), jnp.float32),
            pltpu.VMEM((D, DH), jnp.float32),
            pltpu.VMEM((D, DH), jnp.float32),
            pltpu.VMEM((D, D), jnp.float32),
            pltpu.VMEM((2, M, D), jnp.float32),
            pltpu.VMEM((2, M, D), jnp.float32),
            pltpu.VMEM((2, M, D), jnp.float32),
            pltpu.SemaphoreType.DMA((2,)),
            pltpu.SemaphoreType.DMA((2,)),
            pltpu.SemaphoreType.DMA((2,)),
            pltpu.SemaphoreType.DMA((2,)),
            pltpu.SemaphoreType.DMA,
        ],
        compiler_params=pltpu.CompilerParams(
            collective_id=0,
            vmem_limit_bytes=63 * 1024 * 1024,
        ),
    )(x, Wq, Wo, Wk, Wv)


# device time: 823319 ns/iter; 1.1447x vs baseline; 1.0043x over previous
import jax
import jax.numpy as jnp
from jax import lax
from jax.experimental import pallas as pl
from jax.experimental.pallas import tpu as pltpu

N_DEV = 16
S = 4096
M = 256
D = 1024
HL = 8
DH = 128
SCALE = 0.08838834764831843


def _bdot(a, b, dims=None):
    a16 = a.astype(jnp.bfloat16)
    b16 = b.astype(jnp.bfloat16)
    if dims is None:
        dims = (((a.ndim - 1,), (0,)), ((), ()))
    return lax.dot_general(a16, b16, dims,
                           preferred_element_type=jnp.float32)


def kernel(x, Wq, Wo, Wk, Wv):
    def body(x_ref, wq_hbm, wo_hbm, wk_hbm, wv_hbm, out_ref,
             x_full, p_acc, k_scr, v_scr, wq_h, wk_h, wv_h, wo_scr,
             comm, rs_send, rs_recv,
             ag_send_sems, ag_recv_sems, rs_send_sems, rs_recv_sems,
             copy_sem):
        my = lax.axis_index("i")
        left = jnp.mod(my - 1, N_DEV)
        right = jnp.mod(my + 1, N_DEV)

        wo_copy = pltpu.make_async_copy(wo_hbm, wo_scr, copy_sem)
        wo_copy.start()
        wo_copy.wait()

        barrier = pltpu.get_barrier_semaphore()
        for nbr in (left, right):
            pl.semaphore_signal(barrier, inc=1, device_id=(nbr,),
                                device_id_type=pl.DeviceIdType.MESH)
        pl.semaphore_wait(barrier, 2)

        x_full[pl.ds(my * M, M), :] = x_ref[0].astype(jnp.bfloat16)
        comm[0] = x_ref[0]

        for h in range(N_DEV - 1):
            s_slot = h % 2
            r_slot = (h + 1) % 2
            rdma = pltpu.make_async_remote_copy(
                src_ref=comm.at[s_slot],
                dst_ref=comm.at[r_slot],
                send_sem=ag_send_sems.at[s_slot],
                recv_sem=ag_recv_sems.at[r_slot],
                device_id=(right,),
                device_id_type=pl.DeviceIdType.MESH,
            )
            rdma.start()
            rdma.wait()
            origin = jnp.mod(my - h - 1, N_DEV)
            x_full[pl.ds(origin * M, M), :] = (
                comm[r_slot].astype(jnp.bfloat16))

        for hh in range(HL):
            for whbm, wscr in ((wq_hbm, wq_h), (wk_hbm, wk_h),
                               (wv_hbm, wv_h)):
                c = pltpu.make_async_copy(
                    whbm.at[:, hh * DH:(hh + 1) * DH], wscr, copy_sem)
                c.start()
                c.wait()

            def kv_block(rb, _):
                xb = x_full[pl.ds(rb * M, M), :]
                k_scr[pl.ds(rb * M, M), :] = (
                    _bdot(xb, wk_h[...]).astype(jnp.bfloat16))
                v_scr[pl.ds(rb * M, M), :] = (
                    _bdot(xb, wv_h[...]).astype(jnp.bfloat16))
                return 0

            lax.fori_loop(0, N_DEV, kv_block, 0)

            def qb_block(qb, _, first=(hh == 0), h0=hh * DH):
                xq = x_full[pl.ds(qb * M, M), :]
                q = _bdot(xq, wq_h[...])
                s = _bdot(q, k_scr[...],
                          (((1,), (1,)), ((), ()))) * SCALE
                m_ = jnp.max(s, axis=-1, keepdims=True)
                p_ = jnp.exp(s - m_)
                l_ = jnp.sum(p_, axis=-1, keepdims=True)
                o = _bdot(p_, v_scr[...]) / l_
                upd = _bdot(o, wo_scr[h0:h0 + DH, :])
                if first:
                    p_acc[pl.ds(qb * M, M), :] = upd
                else:
                    p_acc[pl.ds(qb * M, M), :] = (
                        p_acc[pl.ds(qb * M, M), :] + upd)
                return 0

            lax.fori_loop(0, N_DEV, qb_block, 0)

        for t in range(N_DEV - 1):
            c = jnp.mod(my - 1 - t, N_DEV)
            pc = p_acc[pl.ds(c * M, M), :]
            if t == 0:
                payload = pc
            else:
                payload = rs_recv[(t - 1) % 2] + pc
            rs_send[t % 2] = payload
            rdma = pltpu.make_async_remote_copy(
                src_ref=rs_send.at[t % 2],
                dst_ref=rs_recv.at[t % 2],
                send_sem=rs_send_sems.at[t % 2],
                recv_sem=rs_recv_sems.at[t % 2],
                device_id=(right,),
                device_id_type=pl.DeviceIdType.MESH,
            )
            rdma.start()
            rdma.wait()

        out_ref[0] = (rs_recv[(N_DEV - 2) % 2]
                      + p_acc[pl.ds(my * M, M), :])

    return pl.pallas_call(
        body,
        out_shape=jax.ShapeDtypeStruct((1, M, D), jnp.float32),
        in_specs=[
            pl.BlockSpec(memory_space=pltpu.VMEM),
            pl.BlockSpec(memory_space=pltpu.MemorySpace.HBM),
            pl.BlockSpec(memory_space=pltpu.MemorySpace.HBM),
            pl.BlockSpec(memory_space=pltpu.MemorySpace.HBM),
            pl.BlockSpec(memory_space=pltpu.MemorySpace.HBM),
        ],
        out_specs=pl.BlockSpec(memory_space=pltpu.VMEM),
        scratch_shapes=[
            pltpu.VMEM((S, D), jnp.bfloat16),
            pltpu.VMEM((S, D), jnp.float32),
            pltpu.VMEM((S, DH), jnp.bfloat16),
            pltpu.VMEM((S, DH), jnp.bfloat16),
            pltpu.VMEM((D, DH), jnp.float32),
            pltpu.VMEM((D, DH), jnp.float32),
            pltpu.VMEM((D, DH), jnp.float32),
            pltpu.VMEM((D, D), jnp.float32),
            pltpu.VMEM((2, M, D), jnp.float32),
            pltpu.VMEM((2, M, D), jnp.float32),
            pltpu.VMEM((2, M, D), jnp.float32),
            pltpu.SemaphoreType.DMA((2,)),
            pltpu.SemaphoreType.DMA((2,)),
            pltpu.SemaphoreType.DMA((2,)),
            pltpu.SemaphoreType.DMA((2,)),
            pltpu.SemaphoreType.DMA,
        ],
        compiler_params=pltpu.CompilerParams(
            collective_id=0,
            vmem_limit_bytes=63 * 1024 * 1024,
        ),
    )(x, Wq, Wo, Wk, Wv)
